# baseline (speedup 1.0000x reference)
import functools

import jax
import jax.numpy as jnp
from jax.experimental import pallas as pl
from jax.experimental.pallas import tpu as pltpu

_VMEM_LIMIT_BYTES = 60000 * 1024


def _upsample2x(prev, H2, W2, C):
    t = jnp.broadcast_to(prev.reshape(H2 * W2, 1, C), (H2 * W2, 2, C))
    t = t.reshape(H2 * 2 * W2, C)
    t = jnp.broadcast_to(t.reshape(H2, 1, 2 * W2, C), (H2, 2, 2 * W2, C))
    return t.reshape(4 * H2 * W2, C)


def _level_body(H, W, x_ref, w1_ref, b1_ref, prev_s, w9_ref, b3_ref,
                lat_s, out_ref, p_ref):
    C = w1_ref.shape[1]
    HW = H * W

    lat = jnp.dot(x_ref[...], w1_ref[...],
                  preferred_element_type=jnp.float32)
    lat = lat + b1_ref[...]
    if prev_s is not None:
        lat = lat + _upsample2x(prev_s[...], H // 2, W // 2, C)
    if lat_s is not None:
        lat_s[...] = lat

    latb = lat.astype(jnp.bfloat16).reshape(H, W, C)
    zrow = jnp.zeros((1, W, 3 * C), jnp.bfloat16)
    zcol = jnp.zeros((H + 2, 8, C), jnp.bfloat16)
    p_ref[0:1, :, :] = zrow
    p_ref[H + 1:H + 2, :, :] = zrow
    p_ref[:, 0:8, 0:C] = zcol
    p_ref[:, W - 8:W, 2 * C:3 * C] = zcol
    p_ref[1:H + 1, 1:W, 0:C] = latb[:, 0:W - 1, :]
    p_ref[1:H + 1, :, C:2 * C] = latb
    p_ref[1:H + 1, 0:W - 1, 2 * C:3 * C] = latb[:, 1:W, :]

    wd = [jnp.concatenate([w9_ref[3 * dy + b] for b in range(3)], axis=0)
          for dy in range(3)]
    RC = max(1, min(H, 512 // W))
    CH = RC * W
    for mc in range(H // RC):
        r0 = mc * RC
        acc = None
        for dy in range(3):
            patch = p_ref[r0 + dy:r0 + dy + RC, :, :].reshape(CH, 3 * C)
            d = jnp.dot(patch, wd[dy], preferred_element_type=jnp.float32)
            acc = d if acc is None else acc + d
        out_ref[mc * CH:(mc + 1) * CH, :] = acc + b3_ref[...]


def _fpn_kernel(dims,
                x5_ref, w15_ref, b15_ref, w95_ref, b35_ref,
                x4_ref, w14_ref, b14_ref, w94_ref, b34_ref,
                x3_ref, w13_ref, b13_ref, w93_ref, b33_ref,
                out3_ref, out4_ref, out5_ref,
                lat5_s, lat4_s, p5_s, p4_s, p3_s):
    (h5, w5), (h4, w4), (h3, w3) = dims
    lvl = pl.program_id(1)

    @pl.when(lvl == 0)
    def _p5():
        _level_body(h5, w5, x5_ref, w15_ref, b15_ref, None, w95_ref, b35_ref,
                    lat5_s, out5_ref, p5_s)

    @pl.when(lvl == 1)
    def _p4():
        _level_body(h4, w4, x4_ref, w14_ref, b14_ref, lat5_s, w94_ref, b34_ref,
                    lat4_s, out4_ref, p4_s)

    @pl.when(lvl == 2)
    def _p3():
        _level_body(h3, w3, x3_ref, w13_ref, b13_ref, lat4_s, w93_ref, b33_ref,
                    None, out3_ref, p3_s)


def kernel(c3, c4, c5,
           p5_1_w, p5_1_b, p5_2_w, p5_2_b,
           p4_1_w, p4_1_b, p4_2_w, p4_2_b,
           p3_1_w, p3_1_b, p3_2_w, p3_2_b):
    N = c3.shape[0]
    C = p5_1_w.shape[1]
    bf = jnp.bfloat16

    def to_rows(x):
        n, c, h, w = x.shape
        return jnp.transpose(x, (0, 2, 3, 1)).astype(bf).reshape(n, h * w, c)

    x5, x4, x3 = to_rows(c5), to_rows(c4), to_rows(c3)
    h5, w5 = c5.shape[2], c5.shape[3]
    h4, w4 = c4.shape[2], c4.shape[3]
    h3, w3_ = c3.shape[2], c3.shape[3]
    dims = ((h5, w5), (h4, w4), (h3, w3_))

    def full(a):
        shape = a.shape
        return pl.BlockSpec(shape, lambda n, s: (0,) * len(shape))

    def batched(a):
        shape = a.shape[1:]
        return pl.BlockSpec((None,) + shape, lambda n, s: (n,) + (0,) * len(shape))

    args = [
        x5, p5_1_w.astype(bf), p5_1_b, p5_2_w.astype(bf), p5_2_b,
        x4, p4_1_w.astype(bf), p4_1_b, p4_2_w.astype(bf), p4_2_b,
        x3, p3_1_w.astype(bf), p3_1_b, p3_2_w.astype(bf), p3_2_b,
    ]
    in_specs = [batched(a) if k % 5 == 0 else full(a)
                for k, a in enumerate(args)]

    out_shape = (jax.ShapeDtypeStruct((N, h3 * w3_, C), jnp.float32),
                 jax.ShapeDtypeStruct((N, h4 * w4, C), jnp.float32),
                 jax.ShapeDtypeStruct((N, h5 * w5, C), jnp.float32))
    out_specs = (pl.BlockSpec((None, h3 * w3_, C), lambda n, s: (n, 0, 0)),
                 pl.BlockSpec((None, h4 * w4, C), lambda n, s: (n, 0, 0)),
                 pl.BlockSpec((None, h5 * w5, C), lambda n, s: (n, 0, 0)))

    res = pl.pallas_call(
        functools.partial(_fpn_kernel, dims),
        grid=(N, 3),
        in_specs=in_specs,
        out_specs=out_specs,
        out_shape=out_shape,
        scratch_shapes=[
            pltpu.VMEM((h5 * w5, C), jnp.float32),
            pltpu.VMEM((h4 * w4, C), jnp.float32),
            pltpu.VMEM((h5 + 2, w5, 3 * C), jnp.bfloat16),
            pltpu.VMEM((h4 + 2, w4, 3 * C), jnp.bfloat16),
            pltpu.VMEM((h3 + 2, w3_, 3 * C), jnp.bfloat16),
        ],
        compiler_params=pltpu.CompilerParams(
            dimension_semantics=("parallel", "arbitrary"),
            vmem_limit_bytes=_VMEM_LIMIT_BYTES,
        ),
    )(*args)

    p3_out, p4_out, p5_out = res

    def to_nchw(o, h, w):
        return jnp.transpose(o.reshape(N, h, w, C), (0, 3, 1, 2))

    return [to_nchw(p3_out, h3, w3_), to_nchw(p4_out, h4, w4),
            to_nchw(p5_out, h5, w5)]


# speedup vs baseline: 1.0678x; 1.0678x over previous
import functools

import jax
import jax.numpy as jnp
from jax.experimental import pallas as pl
from jax.experimental.pallas import tpu as pltpu

_VMEM_LIMIT_BYTES = 60000 * 1024


def _upsample2x(prev, H2, W2, C):
    t = jnp.broadcast_to(prev.reshape(H2 * W2, 1, C), (H2 * W2, 2, C))
    t = t.reshape(H2 * 2 * W2, C)
    t = jnp.broadcast_to(t.reshape(H2, 1, 2 * W2, C), (H2, 2, 2 * W2, C))
    return t.reshape(4 * H2 * W2, C)


def _level_body(H, W, x_ref, w1_ref, b1_ref, prev_s, w9_ref, b3_ref,
                lat_s, out_ref, p_ref):
    C = w1_ref.shape[1]
    HW = H * W

    lat = jnp.dot(x_ref[...], w1_ref[...].astype(jnp.bfloat16),
                  preferred_element_type=jnp.float32)
    lat = lat + b1_ref[...]
    if prev_s is not None:
        lat = lat + _upsample2x(prev_s[...], H // 2, W // 2, C)
    if lat_s is not None:
        lat_s[...] = lat

    latb = lat.astype(jnp.bfloat16).reshape(H, W, C)
    zrow = jnp.zeros((1, W, 3 * C), jnp.bfloat16)
    zcol = jnp.zeros((H + 2, 8, C), jnp.bfloat16)
    p_ref[0:1, :, :] = zrow
    p_ref[H + 1:H + 2, :, :] = zrow
    p_ref[:, 0:8, 0:C] = zcol
    p_ref[:, W - 8:W, 2 * C:3 * C] = zcol
    p_ref[1:H + 1, 1:W, 0:C] = latb[:, 0:W - 1, :]
    p_ref[1:H + 1, :, C:2 * C] = latb
    p_ref[1:H + 1, 0:W - 1, 2 * C:3 * C] = latb[:, 1:W, :]

    wd = [jnp.concatenate([w9_ref[3 * dy + b].astype(jnp.bfloat16)
                           for b in range(3)], axis=0) for dy in range(3)]
    RC = max(1, min(H, 512 // W))
    CH = RC * W
    for mc in range(H // RC):
        r0 = mc * RC
        acc = None
        for dy in range(3):
            patch = p_ref[r0 + dy:r0 + dy + RC, :, :].reshape(CH, 3 * C)
            d = jnp.dot(patch, wd[dy], preferred_element_type=jnp.float32)
            acc = d if acc is None else acc + d
        out_ref[mc * CH:(mc + 1) * CH, :] = acc + b3_ref[...]


def _fpn_kernel(dims,
                x5_ref, w15_ref, b15_ref, w95_ref, b35_ref,
                x4_ref, w14_hbm, b14_ref, w94_hbm, b34_ref,
                x3_ref, w13_hbm, b13_ref, w93_hbm, b33_ref,
                out3_ref, out4_ref, out5_ref,
                lat5_s, lat4_s, p5_s, p4_s, p3_s,
                w14_v, w94_v, w13_v, w93_v, sem4, sem3):
    (h5, w5), (h4, w4), (h3, w3) = dims
    lvl = pl.program_id(1)

    cps4 = [pltpu.make_async_copy(w14_hbm, w14_v, sem4),
            pltpu.make_async_copy(w94_hbm, w94_v, sem4)]
    cps3 = [pltpu.make_async_copy(w13_hbm, w13_v, sem3),
            pltpu.make_async_copy(w93_hbm, w93_v, sem3)]

    @pl.when(lvl == 0)
    def _p5():
        for cp in cps4 + cps3:
            cp.start()
        _level_body(h5, w5, x5_ref, w15_ref, b15_ref, None, w95_ref, b35_ref,
                    lat5_s, out5_ref, p5_s)

    @pl.when(lvl == 1)
    def _p4():
        for cp in cps4:
            cp.wait()
        _level_body(h4, w4, x4_ref, w14_v, b14_ref, lat5_s, w94_v, b34_ref,
                    lat4_s, out4_ref, p4_s)

    @pl.when(lvl == 2)
    def _p3():
        for cp in cps3:
            cp.wait()
        _level_body(h3, w3, x3_ref, w13_v, b13_ref, lat4_s, w93_v, b33_ref,
                    None, out3_ref, p3_s)


def kernel(c3, c4, c5,
           p5_1_w, p5_1_b, p5_2_w, p5_2_b,
           p4_1_w, p4_1_b, p4_2_w, p4_2_b,
           p3_1_w, p3_1_b, p3_2_w, p3_2_b):
    N = c3.shape[0]
    C = p5_1_w.shape[1]
    bf = jnp.bfloat16

    def to_rows(x):
        n, c, h, w = x.shape
        return jnp.transpose(x, (0, 2, 3, 1)).astype(bf).reshape(n, h * w, c)

    x5, x4, x3 = to_rows(c5), to_rows(c4), to_rows(c3)
    h5, w5 = c5.shape[2], c5.shape[3]
    h4, w4 = c4.shape[2], c4.shape[3]
    h3, w3_ = c3.shape[2], c3.shape[3]
    dims = ((h5, w5), (h4, w4), (h3, w3_))

    def full(a):
        shape = a.shape
        return pl.BlockSpec(shape, lambda n, s: (0,) * len(shape))

    def batched(a):
        shape = a.shape[1:]
        return pl.BlockSpec((None,) + shape, lambda n, s: (n,) + (0,) * len(shape))

    args = [
        x5, p5_1_w, p5_1_b, p5_2_w, p5_2_b,
        x4, p4_1_w, p4_1_b, p4_2_w, p4_2_b,
        x3, p3_1_w, p3_1_b, p3_2_w, p3_2_b,
    ]
    in_specs = [pl.BlockSpec(memory_space=pl.ANY) if k in (6, 8, 11, 13)
                else (batched(a) if k % 5 == 0 else full(a))
                for k, a in enumerate(args)]

    out_shape = (jax.ShapeDtypeStruct((N, h3 * w3_, C), jnp.float32),
                 jax.ShapeDtypeStruct((N, h4 * w4, C), jnp.float32),
                 jax.ShapeDtypeStruct((N, h5 * w5, C), jnp.float32))
    out_specs = (pl.BlockSpec((None, h3 * w3_, C), lambda n, s: (n, 0, 0)),
                 pl.BlockSpec((None, h4 * w4, C), lambda n, s: (n, 0, 0)),
                 pl.BlockSpec((None, h5 * w5, C), lambda n, s: (n, 0, 0)))

    res = pl.pallas_call(
        functools.partial(_fpn_kernel, dims),
        grid=(N, 3),
        in_specs=in_specs,
        out_specs=out_specs,
        out_shape=out_shape,
        scratch_shapes=[
            pltpu.VMEM((h5 * w5, C), jnp.float32),
            pltpu.VMEM((h4 * w4, C), jnp.float32),
            pltpu.VMEM((h5 + 2, w5, 3 * C), jnp.bfloat16),
            pltpu.VMEM((h4 + 2, w4, 3 * C), jnp.bfloat16),
            pltpu.VMEM((h3 + 2, w3_, 3 * C), jnp.bfloat16),
            pltpu.VMEM(p4_1_w.shape, jnp.float32),
            pltpu.VMEM(p4_2_w.shape, jnp.float32),
            pltpu.VMEM(p3_1_w.shape, jnp.float32),
            pltpu.VMEM(p3_2_w.shape, jnp.float32),
            pltpu.SemaphoreType.DMA,
            pltpu.SemaphoreType.DMA,
        ],
        compiler_params=pltpu.CompilerParams(
            dimension_semantics=("parallel", "arbitrary"),
            vmem_limit_bytes=_VMEM_LIMIT_BYTES,
        ),
    )(*args)

    p3_out, p4_out, p5_out = res

    def to_nchw(o, h, w):
        return jnp.transpose(o.reshape(N, h, w, C), (0, 3, 1, 2))

    return [to_nchw(p3_out, h3, w3_), to_nchw(p4_out, h4, w4),
            to_nchw(p5_out, h5, w5)]


# speedup vs baseline: 1.1090x; 1.0386x over previous
import functools

import jax
import jax.numpy as jnp
from jax.experimental import pallas as pl
from jax.experimental.pallas import tpu as pltpu

_VMEM_LIMIT_BYTES = 60000 * 1024


def _upsample2x(prev, H2, W2, C):
    t = jnp.broadcast_to(prev.reshape(H2 * W2, 1, C), (H2 * W2, 2, C))
    t = t.reshape(H2 * 2 * W2, C)
    t = jnp.broadcast_to(t.reshape(H2, 1, 2 * W2, C), (H2, 2, 2 * W2, C))
    return t.reshape(4 * H2 * W2, C)


def _level_body(H, W, x_ref, w1_ref, b1_ref, prev_s, w9_ref, b3_ref,
                lat_s, out_ref, p_ref):
    C = w1_ref.shape[1]
    HW = H * W

    lat = jnp.dot(x_ref[...], w1_ref[...].astype(jnp.bfloat16),
                  preferred_element_type=jnp.float32)
    lat = lat + b1_ref[...]
    if prev_s is not None:
        lat = lat + _upsample2x(prev_s[...], H // 2, W // 2, C)
    if lat_s is not None:
        lat_s[...] = lat

    latb = lat.astype(jnp.bfloat16).reshape(H, W, C)
    zrow = jnp.zeros((1, W, 3 * C), jnp.bfloat16)
    zcol = jnp.zeros((H + 2, 8, C), jnp.bfloat16)
    p_ref[0:1, :, :] = zrow
    p_ref[H + 1:H + 2, :, :] = zrow
    p_ref[:, 0:8, 0:C] = zcol
    p_ref[:, W - 8:W, 2 * C:3 * C] = zcol
    p_ref[1:H + 1, 1:W, 0:C] = latb[:, 0:W - 1, :]
    p_ref[1:H + 1, :, C:2 * C] = latb
    p_ref[1:H + 1, 0:W - 1, 2 * C:3 * C] = latb[:, 1:W, :]

    wd = [jnp.concatenate([w9_ref[3 * dy + b].astype(jnp.bfloat16)
                           for b in range(3)], axis=0) for dy in range(3)]
    RC = max(1, min(H, 512 // W))
    CH = RC * W
    for mc in range(H // RC):
        r0 = mc * RC
        acc = None
        for dy in range(3):
            patch = p_ref[r0 + dy:r0 + dy + RC, :, :].reshape(CH, 3 * C)
            d = jnp.dot(patch, wd[dy], preferred_element_type=jnp.float32)
            acc = d if acc is None else acc + d
        out_ref[mc * CH:(mc + 1) * CH, :] = acc + b3_ref[...]


def _fpn_kernel(dims,
                x5_ref, w15_ref, b15_ref, w95_ref, b35_ref,
                x4_ref, w14_ref, b14_ref, w94_ref, b34_ref,
                x3_ref, w13_ref, b13_ref, w93_ref, b33_ref,
                out3_ref, out4_ref, out5_ref,
                lat5_s, lat4_s, p5_s, p4_s, p3_s):
    (h5, w5), (h4, w4), (h3, w3) = dims
    lvl = pl.program_id(1)

    @pl.when(lvl == 0)
    def _p5():
        _level_body(h5, w5, x5_ref, w15_ref, b15_ref, None, w95_ref, b35_ref,
                    lat5_s, out5_ref, p5_s)

    @pl.when(lvl == 1)
    def _p4():
        _level_body(h4, w4, x4_ref, w14_ref, b14_ref, lat5_s, w94_ref, b34_ref,
                    lat4_s, out4_ref, p4_s)

    @pl.when(lvl == 2)
    def _p3():
        _level_body(h3, w3, x3_ref, w13_ref, b13_ref, lat4_s, w93_ref, b33_ref,
                    None, out3_ref, p3_s)


def kernel(c3, c4, c5,
           p5_1_w, p5_1_b, p5_2_w, p5_2_b,
           p4_1_w, p4_1_b, p4_2_w, p4_2_b,
           p3_1_w, p3_1_b, p3_2_w, p3_2_b):
    N = c3.shape[0]
    C = p5_1_w.shape[1]
    bf = jnp.bfloat16

    def to_rows(x):
        n, c, h, w = x.shape
        return jnp.transpose(x, (0, 2, 3, 1)).astype(bf).reshape(n, h * w, c)

    x5, x4, x3 = to_rows(c5), to_rows(c4), to_rows(c3)
    h5, w5 = c5.shape[2], c5.shape[3]
    h4, w4 = c4.shape[2], c4.shape[3]
    h3, w3_ = c3.shape[2], c3.shape[3]
    dims = ((h5, w5), (h4, w4), (h3, w3_))

    def full(a):
        shape = a.shape
        return pl.BlockSpec(shape, lambda n, s: (0,) * len(shape))

    def batched(a):
        shape = a.shape[1:]
        return pl.BlockSpec((None,) + shape, lambda n, s: (n,) + (0,) * len(shape))

    args = [
        x5, p5_1_w, p5_1_b, p5_2_w, p5_2_b,
        x4, p4_1_w, p4_1_b, p4_2_w, p4_2_b,
        x3, p3_1_w, p3_1_b, p3_2_w, p3_2_b,
    ]
    in_specs = [batched(a) if k % 5 == 0 else full(a)
                for k, a in enumerate(args)]

    out_shape = (jax.ShapeDtypeStruct((N, h3 * w3_, C), jnp.float32),
                 jax.ShapeDtypeStruct((N, h4 * w4, C), jnp.float32),
                 jax.ShapeDtypeStruct((N, h5 * w5, C), jnp.float32))
    out_specs = (pl.BlockSpec((None, h3 * w3_, C), lambda n, s: (n, 0, 0)),
                 pl.BlockSpec((None, h4 * w4, C), lambda n, s: (n, 0, 0)),
                 pl.BlockSpec((None, h5 * w5, C), lambda n, s: (n, 0, 0)))

    res = pl.pallas_call(
        functools.partial(_fpn_kernel, dims),
        grid=(N, 3),
        in_specs=in_specs,
        out_specs=out_specs,
        out_shape=out_shape,
        scratch_shapes=[
            pltpu.VMEM((h5 * w5, C), jnp.float32),
            pltpu.VMEM((h4 * w4, C), jnp.float32),
            pltpu.VMEM((h5 + 2, w5, 3 * C), jnp.bfloat16),
            pltpu.VMEM((h4 + 2, w4, 3 * C), jnp.bfloat16),
            pltpu.VMEM((h3 + 2, w3_, 3 * C), jnp.bfloat16),
        ],
        compiler_params=pltpu.CompilerParams(
            dimension_semantics=("parallel", "arbitrary"),
            vmem_limit_bytes=_VMEM_LIMIT_BYTES,
        ),
    )(*args)

    p3_out, p4_out, p5_out = res

    def to_nchw(o, h, w):
        return jnp.transpose(o.reshape(N, h, w, C), (0, 3, 1, 2))

    return [to_nchw(p3_out, h3, w3_), to_nchw(p4_out, h4, w4),
            to_nchw(p5_out, h5, w5)]
